# Initial kernel scaffold; baseline (speedup 1.0000x reference)
import jax
import jax.numpy as jnp
from jax.experimental import pallas as pl


def kernel(el_ids, property_values, elements_in_region, value):
    raise NotImplementedError("write your pallas kernel here")


# speedup vs baseline: 3.7812x; 3.7812x over previous
import functools

import jax
import jax.numpy as jnp
from jax import lax
from jax.experimental import pallas as pl
from jax.experimental.pallas import tpu as pltpu, tpu_sc as plsc

NELEM = 1_000_000
NUM_IDS = 16_384
NUM_REGIONS_PAD = 64
NC, NS, L = 2, 16, 16
NW = NC * NS
CHUNK = 31_256
TAIL = NELEM - 31 * CHUNK
IDS_PER_TILE = NUM_IDS // NS

_MESH = plsc.VectorSubcoreMesh(core_axis_name="c", subcore_axis_name="s")


@functools.partial(
    pl.kernel,
    out_type=jax.ShapeDtypeStruct((NELEM,), jnp.float32),
    mesh=_MESH,
    scratch_types=[
        pltpu.VMEM((CHUNK,), jnp.float32),
        pltpu.VMEM((IDS_PER_TILE,), jnp.int32),
        pltpu.VMEM((NUM_REGIONS_PAD,), jnp.int32),
        pltpu.VMEM((NS, NUM_REGIONS_PAD), jnp.int32),
        pltpu.VMEM((NUM_REGIONS_PAD,), jnp.int32),
        pltpu.VMEM((L,), jnp.float32),
        pltpu.VMEM_SHARED((NS, NUM_REGIONS_PAD), jnp.int32),
    ],
    compiler_params=pltpu.CompilerParams(needs_layout_passes=False),
)
def _region_set_kernel(ids_hbm, eir_hbm, val_hbm, prop_hbm, out_hbm,
                       vbuf, ids_v, hits_v, allhits_v, eir_v, val_v, sh_hits):
    c = lax.axis_index("c")
    s = lax.axis_index("s")
    wid = s * NC + c

    pltpu.sync_copy(ids_hbm.at[pl.ds(s * IDS_PER_TILE, IDS_PER_TILE)], ids_v)
    zero16 = jnp.zeros((L,), jnp.int32)
    one16 = jnp.full((L,), 1, jnp.int32)
    for k in range(NUM_REGIONS_PAD // L):
        hits_v[pl.ds(k * L, L)] = zero16
    for k in range(IDS_PER_TILE // L):
        idv = ids_v[pl.ds(k * L, L)]
        plsc.store_scatter(hits_v, [idv], one16)
    pltpu.sync_copy(hits_v, sh_hits.at[s])
    plsc.subcore_barrier()
    pltpu.sync_copy(sh_hits, allhits_v)

    hit = []
    for k in range(NUM_REGIONS_PAD // L):
        h = allhits_v[0, pl.ds(k * L, L)]
        for r in range(1, NS):
            h = jnp.maximum(h, allhits_v[r, pl.ds(k * L, L)])
        hit.append(h)

    pltpu.sync_copy(eir_hbm, eir_v)
    pltpu.sync_copy(val_hbm, val_v)
    vval = val_v[...]

    base = pl.multiple_of(wid * CHUNK, 8)

    def copy_scatter_chunk(size):
        pltpu.sync_copy(prop_hbm.at[pl.ds(base, size)], vbuf.at[pl.ds(0, size)])
        for k in range(NUM_REGIONS_PAD // L):
            t = eir_v[pl.ds(k * L, L)]
            m = (hit[k] > 0) & (t >= base) & (t < base + size)
            local = jnp.where(m, t - base, 0)
            plsc.store_scatter(vbuf, [local], vval, mask=m)
        pltpu.sync_copy(vbuf.at[pl.ds(0, size)], out_hbm.at[pl.ds(base, size)])

    @pl.when(wid < NW - 1)
    def _():
        copy_scatter_chunk(CHUNK)

    @pl.when(wid == NW - 1)
    def _():
        copy_scatter_chunk(TAIL)


def kernel(el_ids, property_values, elements_in_region, value):
    ids = el_ids.astype(jnp.int32)
    eir_p = jnp.pad(elements_in_region.astype(jnp.int32),
                    (0, NUM_REGIONS_PAD - elements_in_region.shape[0]),
                    constant_values=-1)
    val16 = jnp.broadcast_to(jnp.asarray(value, jnp.float32), (L,))
    return _region_set_kernel(ids, eir_p, val16, property_values)


# speedup vs baseline: 3.8658x; 1.0224x over previous
import functools

import jax
import jax.numpy as jnp
from jax import lax
from jax.experimental import pallas as pl
from jax.experimental.pallas import tpu as pltpu, tpu_sc as plsc

NELEM = 1_000_000
NUM_IDS = 16_384
NUM_REGIONS_PAD = 64
NC, NS, L = 2, 16, 16
NW = NC * NS
NSUB = 4
SUB = 7_816
CHUNK = NSUB * SUB
SUB_T = 7_704
TAIL = NSUB * SUB_T
IDS_PER_TILE = NUM_IDS // NS

_MESH = plsc.VectorSubcoreMesh(core_axis_name="c", subcore_axis_name="s")


@functools.partial(
    pl.kernel,
    out_type=jax.ShapeDtypeStruct((NELEM,), jnp.float32),
    mesh=_MESH,
    scratch_types=[
        pltpu.VMEM((CHUNK,), jnp.float32),
        pltpu.VMEM((IDS_PER_TILE,), jnp.int32),
        pltpu.VMEM((NUM_REGIONS_PAD,), jnp.int32),
        pltpu.VMEM((NS, NUM_REGIONS_PAD), jnp.int32),
        pltpu.VMEM((NUM_REGIONS_PAD,), jnp.int32),
        pltpu.VMEM((L,), jnp.float32),
        pltpu.VMEM_SHARED((NS, NUM_REGIONS_PAD), jnp.int32),
        pltpu.SemaphoreType.DMA,
        pltpu.SemaphoreType.DMA,
    ],
    compiler_params=pltpu.CompilerParams(needs_layout_passes=False),
)
def _region_set_kernel(ids_hbm, eir_hbm, val_hbm, prop_hbm, out_hbm,
                       vbuf, ids_v, hits_v, allhits_v, eir_v, val_v,
                       sh_hits, in_sem, out_sem):
    c = lax.axis_index("c")
    s = lax.axis_index("s")
    wid = s * NC + c
    base = pl.multiple_of(wid * CHUNK, 8)
    last = wid == NW - 1

    def in_copy(j, sub):
        return pltpu.make_async_copy(
            prop_hbm.at[pl.ds(base + j * sub, sub)],
            vbuf.at[pl.ds(j * sub, sub)], in_sem)

    def out_copy(j, sub):
        return pltpu.make_async_copy(
            vbuf.at[pl.ds(j * sub, sub)],
            out_hbm.at[pl.ds(base + j * sub, sub)], out_sem)

    @pl.when(jnp.logical_not(last))
    def _():
        for j in range(NSUB):
            in_copy(j, SUB).start()

    @pl.when(last)
    def _():
        for j in range(NSUB):
            in_copy(j, SUB_T).start()

    pltpu.sync_copy(ids_hbm.at[pl.ds(s * IDS_PER_TILE, IDS_PER_TILE)], ids_v)
    zero16 = jnp.zeros((L,), jnp.int32)
    one16 = jnp.full((L,), 1, jnp.int32)
    for k in range(NUM_REGIONS_PAD // L):
        hits_v[pl.ds(k * L, L)] = zero16
    for k in range(IDS_PER_TILE // L):
        idv = ids_v[pl.ds(k * L, L)]
        plsc.store_scatter(hits_v, [idv], one16)
    pltpu.sync_copy(hits_v, sh_hits.at[s])
    plsc.subcore_barrier()
    pltpu.sync_copy(sh_hits, allhits_v)

    hit = []
    for k in range(NUM_REGIONS_PAD // L):
        h = allhits_v[0, pl.ds(k * L, L)]
        for r in range(1, NS):
            h = jnp.maximum(h, allhits_v[r, pl.ds(k * L, L)])
        hit.append(h)

    pltpu.sync_copy(eir_hbm, eir_v)
    pltpu.sync_copy(val_hbm, val_v)
    vval = val_v[...]

    def drain_patch_store(sub):
        for j in range(NSUB):
            in_copy(j, sub).wait()
            sb = base + j * sub
            for k in range(NUM_REGIONS_PAD // L):
                t = eir_v[pl.ds(k * L, L)]
                m = (hit[k] > 0) & (t >= sb) & (t < sb + sub)
                local = jnp.where(m, t - base, 0)
                plsc.store_scatter(vbuf, [local], vval, mask=m)
            out_copy(j, sub).start()
        for j in range(NSUB):
            out_copy(j, sub).wait()

    @pl.when(jnp.logical_not(last))
    def _():
        drain_patch_store(SUB)

    @pl.when(last)
    def _():
        drain_patch_store(SUB_T)


def kernel(el_ids, property_values, elements_in_region, value):
    ids = el_ids.astype(jnp.int32)
    eir_p = jnp.pad(elements_in_region.astype(jnp.int32),
                    (0, NUM_REGIONS_PAD - elements_in_region.shape[0]),
                    constant_values=-1)
    val16 = jnp.broadcast_to(jnp.asarray(value, jnp.float32), (L,))
    return _region_set_kernel(ids, eir_p, val16, property_values)
